# baseline (speedup 1.0000x reference)
import functools

import jax
import jax.numpy as jnp
from jax import lax
from jax.experimental import pallas as pl
from jax.experimental.pallas import tpu as pltpu
from jax.experimental.pallas import tpu_sc as plsc

N = 10000
NPAD = 10112
E = 320000
H = 128
NC = 2
NS = 16
NW = NC * NS
EPW = E // NW
CH = 40
NCHUNK = EPW // CH
NPAIR = NCHUNK // 2 - 1
RPT = NPAD // NS
HG = H // 16

_mesh = plsc.VectorSubcoreMesh(
    core_axis_name="c", subcore_axis_name="s", num_cores=NC, num_subcores=NS)
_params = pltpu.CompilerParams(needs_layout_passes=False,
                               use_tc_tiling_on_sc=False)


CW = 16


@functools.partial(
    pl.kernel,
    out_type=(jax.ShapeDtypeStruct((NW, NCHUNK, CH), jnp.float32),
              jax.ShapeDtypeStruct((NC, NPAD, CW), jnp.float32)),
    mesh=_mesh,
    scratch_types=[
        pltpu.VMEM_SHARED((NPAD, CW), jnp.float32),
        pltpu.VMEM((3 * N,), jnp.float32),
        pltpu.VMEM((8, CH), jnp.int32),
        pltpu.VMEM((8, CH), jnp.int32),
        pltpu.VMEM((CH, CW), jnp.float32),
        pltpu.VMEM((CH,), jnp.float32),
        pltpu.VMEM((CH,), jnp.float32),
        pltpu.SemaphoreType.DMA,
        pltpu.SemaphoreType.DMA,
        pltpu.SemaphoreType.DMA,
        pltpu.SemaphoreType.DMA,
    ],
    compiler_params=_params,
)
def _d2_pass(posf_hbm, pk_hbm, ones_hbm, zeros_hbm,
             d2_hbm, cnt_hbm, cnt_sh, posv, pkvA, pkvB, onesv, ovA, ovB,
             sOutA, sOutB, sCntA, sCntB):
    c = lax.axis_index("c")
    s = lax.axis_index("s")
    wid = c * NS + s
    pltpu.sync_copy(zeros_hbm, cnt_sh.at[pl.ds(s * RPT, RPT)])
    pltpu.sync_copy(posf_hbm, posv)
    pltpu.sync_copy(ones_hbm, onesv)
    plsc.subcore_barrier()

    def compute(pkv, ov):
        for off in (0, 16, 24):
            s16 = pkv[0, pl.ds(off, 16)] * 3
            d16 = pkv[1, pl.ds(off, 16)] * 3
            dx = plsc.load_gather(posv, [s16]) - plsc.load_gather(posv, [d16])
            dy = plsc.load_gather(posv, [s16 + 1]) - plsc.load_gather(posv, [d16 + 1])
            dz = plsc.load_gather(posv, [s16 + 2]) - plsc.load_gather(posv, [d16 + 2])
            ov[pl.ds(off, 16)] = dx * dx + dy * dy + dz * dz

    def issue_outs(i, pkv, ov, sOut, sCnt):
        pltpu.async_copy(ov, d2_hbm.at[wid, i], sOut)
        pltpu.async_copy(onesv, cnt_sh.at[pkv.at[1]], sCnt, add=True)

    def drain_outs(i, pkv, ov, sOut, sCnt):
        pltpu.make_async_copy(ov, d2_hbm.at[wid, i], sOut).wait()
        pltpu.make_async_copy(onesv, cnt_sh.at[pkv.at[1]], sCnt).wait()

    pltpu.sync_copy(pk_hbm.at[wid, 0], pkvA)

    def pair(t, carry):
        i0 = t * 2
        @pl.when(t > 0)
        def _():
            drain_outs(i0 - 1, pkvB, ovB, sOutB, sCntB)
        pltpu.sync_copy(pk_hbm.at[wid, i0 + 1], pkvB)
        compute(pkvA, ovA)
        issue_outs(i0, pkvA, ovA, sOutA, sCntA)
        compute(pkvB, ovB)
        issue_outs(i0 + 1, pkvB, ovB, sOutB, sCntB)
        drain_outs(i0, pkvA, ovA, sOutA, sCntA)
        pltpu.sync_copy(pk_hbm.at[wid, i0 + 2], pkvA)
        return carry

    lax.fori_loop(0, NCHUNK // 2 - 1, pair, 0)

    drain_outs(NCHUNK - 3, pkvB, ovB, sOutB, sCntB)
    pltpu.sync_copy(pk_hbm.at[wid, NCHUNK - 1], pkvB)
    compute(pkvA, ovA)
    issue_outs(NCHUNK - 2, pkvA, ovA, sOutA, sCntA)
    compute(pkvB, ovB)
    pltpu.sync_copy(ovB, d2_hbm.at[wid, NCHUNK - 1])
    pltpu.sync_copy(onesv, cnt_sh.at[pkvB.at[1]], add=True)
    drain_outs(NCHUNK - 2, pkvA, ovA, sOutA, sCntA)

    plsc.subcore_barrier()
    pltpu.sync_copy(cnt_sh.at[pl.ds(s * RPT, RPT)],
                    cnt_hbm.at[c, pl.ds(s * RPT, RPT)])


@functools.partial(
    pl.kernel,
    out_type=jax.ShapeDtypeStruct((NC, NPAD, H), jnp.float32),
    mesh=_mesh,
    scratch_types=[
        pltpu.VMEM_SHARED((NPAD, H), jnp.float32),
        pltpu.VMEM((8, CH), jnp.int32),
        pltpu.VMEM((8, CH), jnp.int32),
        pltpu.VMEM((CH, H), jnp.float32),
        pltpu.VMEM((CH, H), jnp.float32),
        pltpu.VMEM((CH, H), jnp.float32),
        pltpu.VMEM((CH, H), jnp.float32),
        pltpu.VMEM((H,), jnp.float32),
        pltpu.SemaphoreType.DMA,
        pltpu.SemaphoreType.DMA,
        pltpu.SemaphoreType.DMA,
        pltpu.SemaphoreType.DMA,
        pltpu.SemaphoreType.DMA,
        pltpu.SemaphoreType.DMA,
    ],
    compiler_params=_params,
)
def _edge_pass(P_hbm, Q_hbm, pk_hbm, dvec_hbm, zeros_hbm,
               out_hbm, S_sh, pkA, pkB, PbA, PbB, RbA, RbB, dvv,
               sPA, sPB, sQA, sQB, sSA, sSB):
    c = lax.axis_index("c")
    s = lax.axis_index("s")
    wid = c * NS + s
    pltpu.sync_copy(zeros_hbm, S_sh.at[pl.ds(s * RPT, RPT)])
    pltpu.sync_copy(dvec_hbm, dvv)
    plsc.subcore_barrier()

    Dg = [dvv[pl.ds(16 * g, 16)] for g in range(HG)]

    def stage(i, pk):
        pltpu.sync_copy(pk_hbm.at[wid, i], pk)

    H0, H1 = 24, CH - 24

    def _gather_descs(pk, Pb, Rb, sP, sQ):
        return (
            pltpu.make_async_copy(P_hbm.at[pk.at[0, pl.ds(0, H0)]],
                                  Pb.at[pl.ds(0, H0)], sP),
            pltpu.make_async_copy(Q_hbm.at[pk.at[1, pl.ds(0, H0)]],
                                  Rb.at[pl.ds(0, H0)], sQ),
            pltpu.make_async_copy(P_hbm.at[pk.at[0, pl.ds(H0, H1)]],
                                  Pb.at[pl.ds(H0, H1)], sP),
            pltpu.make_async_copy(Q_hbm.at[pk.at[1, pl.ds(H0, H1)]],
                                  Rb.at[pl.ds(H0, H1)], sQ),
        )

    def issue_gathers(pk, Pb, Rb, sP, sQ):
        for d in _gather_descs(pk, Pb, Rb, sP, sQ):
            d.start()

    def wait_gathers(pk, Pb, Rb, sP, sQ):
        for d in _gather_descs(pk, Pb, Rb, sP, sQ):
            d.wait()

    def _rows16(pk, Pb, Rb, base, lanes):
        d16 = plsc.bitcast(pk[2, pl.ds(base, 16)], jnp.float32)
        for r in lanes:
            row = base + r
            dj = jnp.full((16,), d16[r], jnp.float32)
            for g in range(HG):
                v = Pb[row, pl.ds(16 * g, 16)] + Rb[row, pl.ds(16 * g, 16)] \
                    + dj * Dg[g]
                Rb[row, pl.ds(16 * g, 16)] = jnp.maximum(v, 0.0)

    def compute(pk, Pb, Rb):
        def body(sb, carry):
            _rows16(pk, Pb, Rb, sb * 16, range(16))
            return carry

        lax.fori_loop(0, 2, body, 0)
        _rows16(pk, Pb, Rb, 24, range(8, 16))

    def scatter_issue(pk, Rb, sS):
        pltpu.async_copy(Rb, S_sh.at[pk.at[1]], sS, add=True)

    def scatter_drain(pk, Rb, sS):
        pltpu.make_async_copy(Rb, S_sh.at[pk.at[1]], sS).wait()

    stage(0, pkA)
    issue_gathers(pkA, PbA, RbA, sPA, sQA)

    def pair(j, carry):
        i0 = j * 2
        @pl.when(j > 0)
        def _():
            scatter_drain(pkB, RbB, sSB)
        stage(i0 + 1, pkB)
        issue_gathers(pkB, PbB, RbB, sPB, sQB)
        wait_gathers(pkA, PbA, RbA, sPA, sQA)
        compute(pkA, PbA, RbA)
        scatter_issue(pkA, RbA, sSA)
        wait_gathers(pkB, PbB, RbB, sPB, sQB)
        compute(pkB, PbB, RbB)
        scatter_issue(pkB, RbB, sSB)
        scatter_drain(pkA, RbA, sSA)
        stage(i0 + 2, pkA)
        issue_gathers(pkA, PbA, RbA, sPA, sQA)
        return carry

    lax.fori_loop(0, NPAIR, pair, 0)

    scatter_drain(pkB, RbB, sSB)
    stage(NCHUNK - 1, pkB)
    issue_gathers(pkB, PbB, RbB, sPB, sQB)
    wait_gathers(pkA, PbA, RbA, sPA, sQA)
    compute(pkA, PbA, RbA)
    scatter_issue(pkA, RbA, sSA)
    wait_gathers(pkB, PbB, RbB, sPB, sQB)
    compute(pkB, PbB, RbB)
    pltpu.sync_copy(RbB, S_sh.at[pkB.at[1]], add=True)
    scatter_drain(pkA, RbA, sSA)

    plsc.subcore_barrier()
    pltpu.sync_copy(S_sh.at[pl.ds(s * RPT, RPT)],
                    out_hbm.at[c, pl.ds(s * RPT, RPT)])


_BR = 632
_GRID = NPAD // _BR
_HP = jax.lax.Precision.HIGHEST


def _full(shape):
    return pl.BlockSpec(shape, lambda i: tuple(0 for _ in shape))


def _rows(width):
    return pl.BlockSpec((_BR, width), lambda i: (i, 0))


def _core(k):
    return pl.BlockSpec((1, _BR, H), lambda i, _k=k: (_k, i, 0))


def _core_cnt(k):
    return pl.BlockSpec((1, _BR, CW), lambda i, _k=k: (_k, i, 0))


def _tc_prep0(zsb, posb, Winb, binb, Ab, Bb, C8b, bm1b, d2b, hb, Pb, Qb, db):
    db[...] = jnp.sqrt(d2b[...] + 1e-12)
    h = jnp.maximum(jnp.dot(zsb[...], Winb[...],
                            preferred_element_type=jnp.float32,
                            precision=_HP) + binb[...], 0.0)
    posc = jnp.dot(posb[...], C8b[...], preferred_element_type=jnp.float32,
                   precision=_HP)
    hb[...] = h
    Pb[...] = jnp.dot(h, Ab[...], preferred_element_type=jnp.float32,
                      precision=_HP) + posc
    Qb[...] = jnp.dot(h, Bb[...], preferred_element_type=jnp.float32,
                      precision=_HP) - posc + bm1b[...]


def _tc_mid(hb, S0b, S1b, c0b, c1b, posb, W2b, bm2b, Ab, Bb, C8b, bm1b,
            hob, Pb, Qb):
    Ssum = S0b[0] + S1b[0]
    cnt = c0b[0] + c1b[0]
    agg = jnp.dot(Ssum, W2b[...], preferred_element_type=jnp.float32,
                  precision=_HP) + cnt[:, 0:1] * bm2b[...]
    hn = hb[...] + jnp.maximum(agg, 0.0)
    posc = jnp.dot(posb[...], C8b[...], preferred_element_type=jnp.float32,
                   precision=_HP)
    hob[...] = hn
    Pb[...] = jnp.dot(hn, Ab[...], preferred_element_type=jnp.float32,
                      precision=_HP) + posc
    Qb[...] = jnp.dot(hn, Bb[...], preferred_element_type=jnp.float32,
                      precision=_HP) - posc + bm1b[...]


def _tc_final(hb, S0b, S1b, c0b, c1b, W2b, bm2b, W3b, b3b, outb):
    Ssum = S0b[0] + S1b[0]
    cnt = c0b[0] + c1b[0]
    agg = jnp.dot(Ssum, W2b[...], preferred_element_type=jnp.float32,
                  precision=_HP) + cnt[:, 0:1] * bm2b[...]
    hn = hb[...] + jnp.maximum(agg, 0.0)
    outb[...] = jnp.dot(hn, W3b[...], preferred_element_type=jnp.float32,
                        precision=_HP) + b3b[...]


def _tc_sqrt(d2b, ob):
    ob[...] = jnp.sqrt(d2b[...] + 1e-12)


def kernel(pos, z, shapes, edge_index, W_in, b_in, Wm1, bm1, Wm2, bm2, W_out, b_out):
    f32 = jnp.float32

    zs8 = jnp.pad(jnp.concatenate([z, shapes], axis=1), ((0, NPAD - N), (0, 5)))
    pos8 = jnp.pad(pos, ((0, NPAD - N), (0, 5)))
    posf = pos.reshape(3 * N)
    srcr = edge_index[0].reshape(NW, NCHUNK, CH)
    dstr = edge_index[1].reshape(NW, NCHUNK, CH)
    zz = jnp.zeros_like(srcr)
    pk0 = jnp.stack([srcr, dstr, zz, zz, zz, zz, zz, zz], axis=2)

    W_in8 = jnp.pad(W_in, ((0, 5), (0, 0)))
    binr = b_in[None, :]
    A = [Wm1[l, 0:H] for l in range(3)]
    B = [Wm1[l, H:2 * H] for l in range(3)]
    C8 = [jnp.pad(Wm1[l, 2 * H:2 * H + 3], ((0, 5), (0, 0))) for l in range(3)]
    dvec = [Wm1[l, 2 * H + 3] for l in range(3)]
    bm1r = [bm1[l][None, :] for l in range(3)]
    bm2r = [bm2[l][None, :] for l in range(3)]
    W3 = jnp.pad(jnp.stack([W_out[:, 0], W_out[:, H], W_out[:, H + 1]], axis=1),
                 ((0, 0), (0, 5)))
    b3 = jnp.pad(jnp.stack([b_out[0], b_out[H], b_out[H + 1]]), (0, 5))[None, :]
    zrows = jnp.zeros((RPT, H), f32)
    zrows_cnt = jnp.zeros((RPT, CW), f32)
    ones80 = jnp.zeros((CH, CW), f32).at[:, 0].set(1.0)

    d2, cnt = _d2_pass(posf, pk0, ones80, zrows_cnt)
    d2p = jnp.pad(d2.reshape(2500, 128), ((0, 60), (0, 0)))

    h, P, Q, dist2d = pl.pallas_call(
        _tc_prep0,
        grid=(_GRID,),
        in_specs=[_rows(8), _rows(8), _full((8, H)), _full((1, H)),
                  _full((H, H)), _full((H, H)), _full((8, H)), _full((1, H)),
                  pl.BlockSpec((160, 128), lambda i: (i, 0))],
        out_specs=[_rows(H), _rows(H), _rows(H),
                   pl.BlockSpec((160, 128), lambda i: (i, 0))],
        out_shape=[jax.ShapeDtypeStruct((NPAD, H), f32)] * 3
        + [jax.ShapeDtypeStruct((2560, 128), f32)],
    )(zs8, pos8, W_in8, binr, A[0], B[0], C8[0], bm1r[0], d2p)
    distr = dist2d[:2500].reshape(NW, NCHUNK, CH)

    pk = pk0.at[:, :, 2, :].set(lax.bitcast_convert_type(distr, jnp.int32))

    for l in range(3):
        S = _edge_pass(P, Q, pk, dvec[l], zrows)
        if l < 2:
            h, P, Q = pl.pallas_call(
                _tc_mid,
                grid=(_GRID,),
                in_specs=[_rows(H), _core(0), _core(1), _core_cnt(0),
                          _core_cnt(1), _rows(8), _full((H, H)), _full((1, H)),
                          _full((H, H)), _full((H, H)), _full((8, H)),
                          _full((1, H))],
                out_specs=[_rows(H), _rows(H), _rows(H)],
                out_shape=[jax.ShapeDtypeStruct((NPAD, H), f32)] * 3,
            )(h, S, S, cnt, cnt, pos8, Wm2[l], bm2r[l],
              A[l + 1], B[l + 1], C8[l + 1], bm1r[l + 1])
        else:
            feats = pl.pallas_call(
                _tc_final,
                grid=(_GRID,),
                in_specs=[_rows(H), _core(0), _core(1), _core_cnt(0),
                          _core_cnt(1), _full((H, H)), _full((1, H)),
                          _full((H, 8)), _full((1, 8))],
                out_specs=_rows(8),
                out_shape=jax.ShapeDtypeStruct((NPAD, 8), f32),
            )(h, S, S, cnt, cnt, Wm2[l], bm2r[l], W3, b3)

    return feats[:N, 0], feats[:N, 1:3]


# speedup vs baseline: 1.0268x; 1.0268x over previous
import functools

import jax
import jax.numpy as jnp
from jax import lax
from jax.experimental import pallas as pl
from jax.experimental.pallas import tpu as pltpu
from jax.experimental.pallas import tpu_sc as plsc

N = 10000
NPAD = 10112
E = 320000
H = 128
NC = 2
NS = 16
NW = NC * NS
EPW = E // NW
CH = 40
NCHUNK = EPW // CH
NPAIR = NCHUNK // 2 - 1
RPT = NPAD // NS
HG = H // 16

_mesh = plsc.VectorSubcoreMesh(
    core_axis_name="c", subcore_axis_name="s", num_cores=NC, num_subcores=NS)
_params = pltpu.CompilerParams(needs_layout_passes=False,
                               use_tc_tiling_on_sc=False)


CW = 16


@functools.partial(
    pl.kernel,
    out_type=(jax.ShapeDtypeStruct((NW, NCHUNK, CH), jnp.float32),
              jax.ShapeDtypeStruct((NC, NPAD, CW), jnp.float32)),
    mesh=_mesh,
    scratch_types=[
        pltpu.VMEM_SHARED((NPAD, CW), jnp.float32),
        pltpu.VMEM((3 * N,), jnp.float32),
        pltpu.VMEM((8, CH), jnp.int32),
        pltpu.VMEM((8, CH), jnp.int32),
        pltpu.VMEM((CH, CW), jnp.float32),
        pltpu.VMEM((CH,), jnp.float32),
        pltpu.VMEM((CH,), jnp.float32),
        pltpu.SemaphoreType.DMA,
        pltpu.SemaphoreType.DMA,
        pltpu.SemaphoreType.DMA,
        pltpu.SemaphoreType.DMA,
    ],
    compiler_params=_params,
)
def _d2_pass(posf_hbm, pk_hbm, ones_hbm, zeros_hbm,
             d2_hbm, cnt_hbm, cnt_sh, posv, pkvA, pkvB, onesv, ovA, ovB,
             sOutA, sOutB, sCntA, sCntB):
    c = lax.axis_index("c")
    s = lax.axis_index("s")
    wid = c * NS + s
    pltpu.sync_copy(zeros_hbm, cnt_sh.at[pl.ds(s * RPT, RPT)])
    pltpu.sync_copy(posf_hbm, posv)
    pltpu.sync_copy(ones_hbm, onesv)
    plsc.subcore_barrier()

    def compute(pkv, ov):
        for off in (0, 16, 24):
            s16 = pkv[0, pl.ds(off, 16)] * 3
            d16 = pkv[1, pl.ds(off, 16)] * 3
            dx = plsc.load_gather(posv, [s16]) - plsc.load_gather(posv, [d16])
            dy = plsc.load_gather(posv, [s16 + 1]) - plsc.load_gather(posv, [d16 + 1])
            dz = plsc.load_gather(posv, [s16 + 2]) - plsc.load_gather(posv, [d16 + 2])
            ov[pl.ds(off, 16)] = dx * dx + dy * dy + dz * dz

    def issue_outs(i, pkv, ov, sOut, sCnt):
        pltpu.async_copy(ov, d2_hbm.at[wid, i], sOut)
        pltpu.async_copy(onesv, cnt_sh.at[pkv.at[1]], sCnt, add=True)

    def drain_outs(i, pkv, ov, sOut, sCnt):
        pltpu.make_async_copy(ov, d2_hbm.at[wid, i], sOut).wait()
        pltpu.make_async_copy(onesv, cnt_sh.at[pkv.at[1]], sCnt).wait()

    pltpu.sync_copy(pk_hbm.at[wid, 0], pkvA)

    def pair(t, carry):
        i0 = t * 2
        @pl.when(t > 0)
        def _():
            drain_outs(i0 - 1, pkvB, ovB, sOutB, sCntB)
        pltpu.sync_copy(pk_hbm.at[wid, i0 + 1], pkvB)
        compute(pkvA, ovA)
        issue_outs(i0, pkvA, ovA, sOutA, sCntA)
        compute(pkvB, ovB)
        issue_outs(i0 + 1, pkvB, ovB, sOutB, sCntB)
        drain_outs(i0, pkvA, ovA, sOutA, sCntA)
        pltpu.sync_copy(pk_hbm.at[wid, i0 + 2], pkvA)
        return carry

    lax.fori_loop(0, NCHUNK // 2 - 1, pair, 0)

    drain_outs(NCHUNK - 3, pkvB, ovB, sOutB, sCntB)
    pltpu.sync_copy(pk_hbm.at[wid, NCHUNK - 1], pkvB)
    compute(pkvA, ovA)
    issue_outs(NCHUNK - 2, pkvA, ovA, sOutA, sCntA)
    compute(pkvB, ovB)
    pltpu.sync_copy(ovB, d2_hbm.at[wid, NCHUNK - 1])
    pltpu.sync_copy(onesv, cnt_sh.at[pkvB.at[1]], add=True)
    drain_outs(NCHUNK - 2, pkvA, ovA, sOutA, sCntA)

    plsc.subcore_barrier()
    pltpu.sync_copy(cnt_sh.at[pl.ds(s * RPT, RPT)],
                    cnt_hbm.at[c, pl.ds(s * RPT, RPT)])


@functools.partial(
    pl.kernel,
    out_type=jax.ShapeDtypeStruct((NC, NPAD, H), jnp.float32),
    mesh=_mesh,
    scratch_types=[
        pltpu.VMEM_SHARED((NPAD, H), jnp.float32),
        pltpu.VMEM((8, CH), jnp.int32),
        pltpu.VMEM((8, CH), jnp.int32),
        pltpu.VMEM((CH, H), jnp.float32),
        pltpu.VMEM((CH, H), jnp.float32),
        pltpu.VMEM((CH, H), jnp.float32),
        pltpu.VMEM((CH, H), jnp.float32),
        pltpu.VMEM((H,), jnp.float32),
        pltpu.SemaphoreType.DMA,
        pltpu.SemaphoreType.DMA,
        pltpu.SemaphoreType.DMA,
        pltpu.SemaphoreType.DMA,
        pltpu.SemaphoreType.DMA,
        pltpu.SemaphoreType.DMA,
    ],
    compiler_params=_params,
)
def _edge_pass(P_hbm, Q_hbm, pk_hbm, dvec_hbm, zeros_hbm,
               out_hbm, S_sh, pkA, pkB, PbA, PbB, RbA, RbB, dvv,
               sPA, sPB, sQA, sQB, sSA, sSB):
    c = lax.axis_index("c")
    s = lax.axis_index("s")
    wid = c * NS + s
    pltpu.sync_copy(zeros_hbm, S_sh.at[pl.ds(s * RPT, RPT)])
    pltpu.sync_copy(dvec_hbm, dvv)
    plsc.subcore_barrier()

    Dg = [dvv[pl.ds(16 * g, 16)] for g in range(HG)]

    def stage(i, pk):
        pltpu.sync_copy(pk_hbm.at[wid, i], pk)

    H0, H1 = 24, CH - 24

    def _gather_descs(pk, Pb, Rb, sP, sQ):
        return (
            pltpu.make_async_copy(P_hbm.at[pk.at[0, pl.ds(0, H0)]],
                                  Pb.at[pl.ds(0, H0)], sP),
            pltpu.make_async_copy(Q_hbm.at[pk.at[1, pl.ds(0, H0)]],
                                  Rb.at[pl.ds(0, H0)], sQ),
            pltpu.make_async_copy(P_hbm.at[pk.at[0, pl.ds(H0, H1)]],
                                  Pb.at[pl.ds(H0, H1)], sP),
            pltpu.make_async_copy(Q_hbm.at[pk.at[1, pl.ds(H0, H1)]],
                                  Rb.at[pl.ds(H0, H1)], sQ),
        )

    def issue_gathers(pk, Pb, Rb, sP, sQ):
        for d in _gather_descs(pk, Pb, Rb, sP, sQ):
            d.start()

    def wait_gathers(pk, Pb, Rb, sP, sQ):
        for d in _gather_descs(pk, Pb, Rb, sP, sQ):
            d.wait()

    def _rows16(pk, Pb, Rb, base, lanes):
        d16 = plsc.bitcast(pk[2, pl.ds(base, 16)], jnp.float32)
        for r in lanes:
            row = base + r
            dj = jnp.full((16,), d16[r], jnp.float32)
            for g in range(HG):
                v = Pb[row, pl.ds(16 * g, 16)] + Rb[row, pl.ds(16 * g, 16)] \
                    + dj * Dg[g]
                Rb[row, pl.ds(16 * g, 16)] = jnp.maximum(v, 0.0)

    def compute(pk, Pb, Rb):
        def body(sb, carry):
            _rows16(pk, Pb, Rb, sb * 16, range(16))
            return carry

        lax.fori_loop(0, 2, body, 0)
        _rows16(pk, Pb, Rb, 24, range(8, 16))

    def scatter_issue(pk, Rb, sS):
        pltpu.async_copy(Rb, S_sh.at[pk.at[1]], sS, add=True)

    def scatter_drain(pk, Rb, sS):
        pltpu.make_async_copy(Rb, S_sh.at[pk.at[1]], sS).wait()

    stage(0, pkA)
    issue_gathers(pkA, PbA, RbA, sPA, sQA)

    def pair(j, carry):
        i0 = j * 2
        @pl.when(j > 0)
        def _():
            scatter_drain(pkB, RbB, sSB)
        stage(i0 + 1, pkB)
        issue_gathers(pkB, PbB, RbB, sPB, sQB)
        wait_gathers(pkA, PbA, RbA, sPA, sQA)
        compute(pkA, PbA, RbA)
        scatter_issue(pkA, RbA, sSA)
        wait_gathers(pkB, PbB, RbB, sPB, sQB)
        compute(pkB, PbB, RbB)
        scatter_issue(pkB, RbB, sSB)
        scatter_drain(pkA, RbA, sSA)
        stage(i0 + 2, pkA)
        issue_gathers(pkA, PbA, RbA, sPA, sQA)
        return carry

    lax.fori_loop(0, NPAIR, pair, 0)

    scatter_drain(pkB, RbB, sSB)
    stage(NCHUNK - 1, pkB)
    issue_gathers(pkB, PbB, RbB, sPB, sQB)
    wait_gathers(pkA, PbA, RbA, sPA, sQA)
    compute(pkA, PbA, RbA)
    scatter_issue(pkA, RbA, sSA)
    wait_gathers(pkB, PbB, RbB, sPB, sQB)
    compute(pkB, PbB, RbB)
    pltpu.sync_copy(RbB, S_sh.at[pkB.at[1]], add=True)
    scatter_drain(pkA, RbA, sSA)

    plsc.subcore_barrier()
    pltpu.sync_copy(S_sh.at[pl.ds(s * RPT, RPT)],
                    out_hbm.at[c, pl.ds(s * RPT, RPT)])


_BR = 632
_GRID = NPAD // _BR
_HP = jax.lax.Precision.HIGHEST


def _full(shape):
    return pl.BlockSpec(shape, lambda i: tuple(0 for _ in shape))


def _rows(width):
    return pl.BlockSpec((_BR, width), lambda i: (i, 0))


def _core(k):
    return pl.BlockSpec((1, _BR, H), lambda i, _k=k: (_k, i, 0))


def _core_cnt(k):
    return pl.BlockSpec((1, _BR, CW), lambda i, _k=k: (_k, i, 0))


def _tc_prep0(zsb, posb, Winb, binb, Ab, Bb, C8b, bm1b, hb, Pb, Qb):
    h = jnp.maximum(jnp.dot(zsb[...], Winb[...],
                            preferred_element_type=jnp.float32,
                            precision=_HP) + binb[...], 0.0)
    posc = jnp.dot(posb[...], C8b[...], preferred_element_type=jnp.float32,
                   precision=_HP)
    hb[...] = h
    Pb[...] = jnp.dot(h, Ab[...], preferred_element_type=jnp.float32,
                      precision=_HP) + posc
    Qb[...] = jnp.dot(h, Bb[...], preferred_element_type=jnp.float32,
                      precision=_HP) - posc + bm1b[...]


def _tc_mid(hb, S0b, S1b, c0b, c1b, posb, W2b, bm2b, Ab, Bb, C8b, bm1b,
            hob, Pb, Qb):
    Ssum = S0b[0] + S1b[0]
    cnt = c0b[0] + c1b[0]
    agg = jnp.dot(Ssum, W2b[...], preferred_element_type=jnp.float32,
                  precision=_HP) + cnt[:, 0:1] * bm2b[...]
    hn = hb[...] + jnp.maximum(agg, 0.0)
    posc = jnp.dot(posb[...], C8b[...], preferred_element_type=jnp.float32,
                   precision=_HP)
    hob[...] = hn
    Pb[...] = jnp.dot(hn, Ab[...], preferred_element_type=jnp.float32,
                      precision=_HP) + posc
    Qb[...] = jnp.dot(hn, Bb[...], preferred_element_type=jnp.float32,
                      precision=_HP) - posc + bm1b[...]


def _tc_final(hb, S0b, S1b, c0b, c1b, W2b, bm2b, W3b, b3b, outb):
    Ssum = S0b[0] + S1b[0]
    cnt = c0b[0] + c1b[0]
    agg = jnp.dot(Ssum, W2b[...], preferred_element_type=jnp.float32,
                  precision=_HP) + cnt[:, 0:1] * bm2b[...]
    hn = hb[...] + jnp.maximum(agg, 0.0)
    outb[...] = jnp.dot(hn, W3b[...], preferred_element_type=jnp.float32,
                        precision=_HP) + b3b[...]


def _tc_sqrt(d2b, ob):
    ob[...] = jnp.sqrt(d2b[...] + 1e-12)


def kernel(pos, z, shapes, edge_index, W_in, b_in, Wm1, bm1, Wm2, bm2, W_out, b_out):
    f32 = jnp.float32

    zs8 = jnp.pad(jnp.concatenate([z, shapes], axis=1), ((0, NPAD - N), (0, 5)))
    pos8 = jnp.pad(pos, ((0, NPAD - N), (0, 5)))
    posf = pos.reshape(3 * N)
    srcr = edge_index[0].reshape(NW, NCHUNK, CH)
    dstr = edge_index[1].reshape(NW, NCHUNK, CH)
    zz = jnp.zeros_like(srcr)
    pk0 = jnp.stack([srcr, dstr, zz, zz, zz, zz, zz, zz], axis=2)

    W_in8 = jnp.pad(W_in, ((0, 5), (0, 0)))
    binr = b_in[None, :]
    A = [Wm1[l, 0:H] for l in range(3)]
    B = [Wm1[l, H:2 * H] for l in range(3)]
    C8 = [jnp.pad(Wm1[l, 2 * H:2 * H + 3], ((0, 5), (0, 0))) for l in range(3)]
    dvec = [Wm1[l, 2 * H + 3] for l in range(3)]
    bm1r = [bm1[l][None, :] for l in range(3)]
    bm2r = [bm2[l][None, :] for l in range(3)]
    W3 = jnp.pad(jnp.stack([W_out[:, 0], W_out[:, H], W_out[:, H + 1]], axis=1),
                 ((0, 0), (0, 5)))
    b3 = jnp.pad(jnp.stack([b_out[0], b_out[H], b_out[H + 1]]), (0, 5))[None, :]
    zrows = jnp.zeros((RPT, H), f32)
    zrows_cnt = jnp.zeros((RPT, CW), f32)
    ones80 = jnp.zeros((CH, CW), f32).at[:, 0].set(1.0)

    d2, cnt = _d2_pass(posf, pk0, ones80, zrows_cnt)
    dist2d = pl.pallas_call(
        _tc_sqrt,
        out_shape=jax.ShapeDtypeStruct((2500, 128), f32),
    )(d2.reshape(2500, 128))
    distr = dist2d.reshape(NW, NCHUNK, CH)

    pk = pk0.at[:, :, 2, :].set(lax.bitcast_convert_type(distr, jnp.int32))

    h, P, Q = pl.pallas_call(
        _tc_prep0,
        grid=(_GRID,),
        in_specs=[_rows(8), _rows(8), _full((8, H)), _full((1, H)),
                  _full((H, H)), _full((H, H)), _full((8, H)), _full((1, H))],
        out_specs=[_rows(H), _rows(H), _rows(H)],
        out_shape=[jax.ShapeDtypeStruct((NPAD, H), f32)] * 3,
    )(zs8, pos8, W_in8, binr, A[0], B[0], C8[0], bm1r[0])

    for l in range(3):
        S = _edge_pass(P, Q, pk, dvec[l], zrows)
        if l < 2:
            h, P, Q = pl.pallas_call(
                _tc_mid,
                grid=(_GRID,),
                in_specs=[_rows(H), _core(0), _core(1), _core_cnt(0),
                          _core_cnt(1), _rows(8), _full((H, H)), _full((1, H)),
                          _full((H, H)), _full((H, H)), _full((8, H)),
                          _full((1, H))],
                out_specs=[_rows(H), _rows(H), _rows(H)],
                out_shape=[jax.ShapeDtypeStruct((NPAD, H), f32)] * 3,
            )(h, S, S, cnt, cnt, pos8, Wm2[l], bm2r[l],
              A[l + 1], B[l + 1], C8[l + 1], bm1r[l + 1])
        else:
            feats = pl.pallas_call(
                _tc_final,
                grid=(_GRID,),
                in_specs=[_rows(H), _core(0), _core(1), _core_cnt(0),
                          _core_cnt(1), _full((H, H)), _full((1, H)),
                          _full((H, 8)), _full((1, 8))],
                out_specs=_rows(8),
                out_shape=jax.ShapeDtypeStruct((NPAD, 8), f32),
            )(h, S, S, cnt, cnt, Wm2[l], bm2r[l], W3, b3)

    return feats[:N, 0], feats[:N, 1:3]
